# Initial kernel scaffold; baseline (speedup 1.0000x reference)
import jax
import jax.numpy as jnp
from jax.experimental import pallas as pl


def kernel(x, ln_scale, w_q, w_k, w_v, w_g, w_r, xl_u, xl_v, codebook, w_o):
    raise NotImplementedError("write your pallas kernel here")


# speedup vs baseline: 2.1769x; 2.1769x over previous
import functools

import jax
import jax.numpy as jnp
import numpy as np
from jax.experimental import pallas as pl

B = 1
T = 2048
C = 128
R = T // C
H = 16
DK = 64
DV = 64
D = 1024
S = 512
TAU = float(DK) ** 0.5
INFTY = 1e30
W = 2 * C

_F32 = jnp.float32


def _sinusoid_np(length, width):
    pos = np.arange(length)[:, None].astype(np.float32)
    i = np.arange(width // 2)[None, :].astype(np.float32)
    freq = np.exp(-(2.0 * i / width) * np.log(10000.0))
    ang = pos * freq
    return np.concatenate([np.sin(ang), np.cos(ang)], axis=-1)


def _proj_kernel(x_ref, s_ref, wq_ref, wk_ref, wv_ref, wg_ref,
                 q_ref, k_ref, v_ref, g_ref):
    x = x_ref[...]
    ms = jnp.mean(jnp.square(x), axis=-1, keepdims=True)
    xt = x * jax.lax.rsqrt(ms + 1e-6) * s_ref[...]
    q_ref[...] = jnp.dot(xt, wq_ref[...], preferred_element_type=_F32)
    k_ref[...] = jnp.dot(xt, wk_ref[...], preferred_element_type=_F32)
    v_ref[...] = jnp.dot(xt, wv_ref[...], preferred_element_type=_F32)
    g_ref[...] = jnp.dot(xt, wg_ref[...], preferred_element_type=_F32)


def _vq_kernel(k_ref, v_ref, q_ref, cb_ref, pe_ref, wr_ref, xlv_ref,
               khat_ref, qr_ref, low_ref, vlag_ref):
    k = k_ref[0]
    cb = cb_ref[0]
    dots = jnp.dot(k, cb.T, preferred_element_type=_F32)
    ksq = jnp.sum(k * k, axis=-1, keepdims=True)
    csq = jnp.sum(cb * cb, axis=-1)
    dist = ksq - 2.0 * dots + csq[None, :]
    dmin = jnp.min(dist, axis=-1, keepdims=True)
    sidx = jax.lax.broadcasted_iota(jnp.int32, (T, S), 1)
    z = jnp.min(jnp.where(dist == dmin, sidx, S), axis=-1, keepdims=True)
    delta = (sidx == z).astype(_F32)
    khat_ref[0] = jnp.dot(delta, cb, preferred_element_type=_F32)

    rprev = jnp.dot(pe_ref[...], wr_ref[0], preferred_element_type=_F32)
    qp = q_ref[0] + xlv_ref[0]
    qr_ref[0] = jnp.dot(qp, rprev.T, preferred_element_type=_F32)

    v = v_ref[0]
    nlist = []
    vlist = []
    for r in range(R):
        dr = delta[r * C:(r + 1) * C]
        vr = v[r * C:(r + 1) * C]
        nlist.append(jnp.sum(dr, axis=0, keepdims=True))
        vlist.append(jax.lax.dot_general(
            dr, vr, (((0,), (0,)), ((), ())),
            preferred_element_type=_F32))
    pn = jnp.zeros((1, S), _F32)
    pv = jnp.zeros((S, DV), _F32)
    for r in range(R):
        low_ref[0, r:r + 1] = pn
        vlag_ref[0, r] = pv
        if 1 <= r < R - 1:
            pn = pn + nlist[r - 1]
            pv = pv + vlist[r - 1]


def _attn_kernel(q_ref, kp_ref, kc_ref, vp_ref, vc_ref, cb_ref,
                 low_ref, vlag_ref, srel_ref, xlu_ref, o_ref):
    r = pl.program_id(1)
    qc = q_ref[0, 0] + xlu_ref[0]
    s_prev = (jnp.dot(qc, kp_ref[0, 0].T, preferred_element_type=_F32)
              + srel_ref[0, 0][:, :C]) / TAU
    s_cur = (jnp.dot(qc, kc_ref[0, 0].T, preferred_element_type=_F32)
             + srel_ref[0, 0][:, C:]) / TAU
    s_cache = jnp.dot(qc, cb_ref[0].T, preferred_element_type=_F32) / TAU

    ci = jax.lax.broadcasted_iota(jnp.int32, (C, C), 0)
    cj = jax.lax.broadcasted_iota(jnp.int32, (C, C), 1)
    s_cur = jnp.where(ci >= cj, s_cur, -INFTY)
    s_prev = jnp.where(r > 0, s_prev, -INFTY)
    low = low_ref[0, 0]
    s_cache = jnp.where(low > 0.0, s_cache, -INFTY)

    m = jnp.maximum(jnp.max(s_prev, axis=-1),
                    jnp.maximum(jnp.max(s_cur, axis=-1),
                                jnp.max(s_cache, axis=-1)))[:, None]
    ap = jnp.exp(s_prev - m)
    ac = jnp.exp(s_cur - m)
    ah = jnp.exp(s_cache - m)
    denom = (jnp.sum(ap, axis=-1) + jnp.sum(ac, axis=-1)
             + jnp.sum(ah * low, axis=-1))[:, None]
    o = (jnp.dot(ap, vp_ref[0, 0], preferred_element_type=_F32)
         + jnp.dot(ac, vc_ref[0, 0], preferred_element_type=_F32)
         + jnp.dot(ah, vlag_ref[0, 0], preferred_element_type=_F32))
    o_ref[0, 0] = o / denom


def _out_kernel(o_ref, g_ref, x_ref, wo_ref, y_ref):
    g = g_ref[...]
    og = o_ref[...] * (g * jax.nn.sigmoid(g))
    y_ref[...] = x_ref[...] + jnp.dot(og, wo_ref[...],
                                      preferred_element_type=_F32)


@jax.jit
def kernel(x, ln_scale, w_q, w_k, w_v, w_g, w_r, xl_u, xl_v, codebook, w_o):
    x2 = x.reshape(T, D)
    lns = ln_scale.reshape(1, D)

    TB = 256
    q, k, v, g = pl.pallas_call(
        _proj_kernel,
        grid=(T // TB,),
        in_specs=[
            pl.BlockSpec((TB, D), lambda i: (i, 0)),
            pl.BlockSpec((1, D), lambda i: (0, 0)),
            pl.BlockSpec((D, H * DK), lambda i: (0, 0)),
            pl.BlockSpec((D, H * DK), lambda i: (0, 0)),
            pl.BlockSpec((D, H * DV), lambda i: (0, 0)),
            pl.BlockSpec((D, H * DV), lambda i: (0, 0)),
        ],
        out_specs=[pl.BlockSpec((TB, H * DK), lambda i: (i, 0))] * 4,
        out_shape=[jax.ShapeDtypeStruct((T, H * DK), _F32)] * 4,
    )(x2, lns, w_q.reshape(D, H * DK), w_k.reshape(D, H * DK),
      w_v.reshape(D, H * DV), w_g.reshape(D, H * DV))

    qh = q.reshape(T, H, DK).transpose(1, 0, 2)
    kh = k.reshape(T, H, DK).transpose(1, 0, 2)
    vh = v.reshape(T, H, DV).transpose(1, 0, 2)
    pe_rev = jnp.asarray(_sinusoid_np(W, D)[::-1].copy())
    wrh = w_r.transpose(1, 0, 2)
    xlv3 = xl_v.reshape(H, 1, DK)
    xlu3 = xl_u.reshape(H, 1, DK)

    khat, qr, lower, vlag = pl.pallas_call(
        _vq_kernel,
        grid=(H,),
        in_specs=[
            pl.BlockSpec((1, T, DK), lambda h: (h, 0, 0)),
            pl.BlockSpec((1, T, DV), lambda h: (h, 0, 0)),
            pl.BlockSpec((1, T, DK), lambda h: (h, 0, 0)),
            pl.BlockSpec((1, S, DK), lambda h: (h, 0, 0)),
            pl.BlockSpec((W, D), lambda h: (0, 0)),
            pl.BlockSpec((1, D, DK), lambda h: (h, 0, 0)),
            pl.BlockSpec((1, 1, DK), lambda h: (h, 0, 0)),
        ],
        out_specs=[
            pl.BlockSpec((1, T, DK), lambda h: (h, 0, 0)),
            pl.BlockSpec((1, T, W), lambda h: (h, 0, 0)),
            pl.BlockSpec((1, R, S), lambda h: (h, 0, 0)),
            pl.BlockSpec((1, R, S, DV), lambda h: (h, 0, 0, 0)),
        ],
        out_shape=[
            jax.ShapeDtypeStruct((H, T, DK), _F32),
            jax.ShapeDtypeStruct((H, T, W), _F32),
            jax.ShapeDtypeStruct((H, R, S), _F32),
            jax.ShapeDtypeStruct((H, R, S, DV), _F32),
        ],
    )(kh, vh, qh, codebook, pe_rev, wrh, xlv3)

    arev = qr.reshape(H, R, C, W)
    arev = jnp.pad(arev, ((0, 0), (0, 0), (0, 0), (0, C - 1)))
    flat = arev.reshape(H, R, C * (3 * C - 1))
    flat = flat[:, :, C - 1:C - 1 + C * (3 * C - 2)]
    srel = flat.reshape(H, R, C, 3 * C - 2)[:, :, :, :W]

    q4 = qh.reshape(H, R, C, DK)
    k4 = khat.reshape(H, R, C, DK)
    v4 = vh.reshape(H, R, C, DV)
    zblk = jnp.zeros((H, 1, C, DK), _F32)
    k4p = jnp.concatenate([zblk, k4[:, :-1]], axis=1)
    v4p = jnp.concatenate([zblk, v4[:, :-1]], axis=1)
    low4 = lower.reshape(H, R, 1, S)

    o = pl.pallas_call(
        _attn_kernel,
        grid=(H, R),
        in_specs=[
            pl.BlockSpec((1, 1, C, DK), lambda h, r: (h, r, 0, 0)),
            pl.BlockSpec((1, 1, C, DK), lambda h, r: (h, r, 0, 0)),
            pl.BlockSpec((1, 1, C, DK), lambda h, r: (h, r, 0, 0)),
            pl.BlockSpec((1, 1, C, DV), lambda h, r: (h, r, 0, 0)),
            pl.BlockSpec((1, 1, C, DV), lambda h, r: (h, r, 0, 0)),
            pl.BlockSpec((1, S, DK), lambda h, r: (h, 0, 0)),
            pl.BlockSpec((1, 1, 1, S), lambda h, r: (h, r, 0, 0)),
            pl.BlockSpec((1, 1, S, DV), lambda h, r: (h, r, 0, 0)),
            pl.BlockSpec((1, 1, C, W), lambda h, r: (h, r, 0, 0)),
            pl.BlockSpec((1, 1, DK), lambda h, r: (h, 0, 0)),
        ],
        out_specs=pl.BlockSpec((1, 1, C, DV), lambda h, r: (h, r, 0, 0)),
        out_shape=jax.ShapeDtypeStruct((H, R, C, DV), _F32),
    )(q4, k4p, k4, v4p, v4, codebook, low4, vlag, srel, xlu3)

    ot = o.transpose(1, 2, 0, 3).reshape(T, H * DV)
    y = pl.pallas_call(
        _out_kernel,
        grid=(T // TB,),
        in_specs=[
            pl.BlockSpec((TB, H * DV), lambda i: (i, 0)),
            pl.BlockSpec((TB, H * DV), lambda i: (i, 0)),
            pl.BlockSpec((TB, D), lambda i: (i, 0)),
            pl.BlockSpec((H * DV, D), lambda i: (0, 0)),
        ],
        out_specs=pl.BlockSpec((TB, D), lambda i: (i, 0)),
        out_shape=jax.ShapeDtypeStruct((T, D), _F32),
    )(ot, g, x2, w_o)

    return y.reshape(B, T, D)


# speedup vs baseline: 3.2396x; 1.4882x over previous
import jax
import jax.numpy as jnp
import numpy as np
from jax.experimental import pallas as pl

B = 1
T = 2048
C = 128
R = T // C
H = 16
DK = 64
DV = 64
D = 1024
S = 512
TAU = float(DK) ** 0.5
INFTY = 1e30
W = 2 * C

_F32 = jnp.float32


def _sinusoid_np(length, width):
    pos = np.arange(length)[:, None].astype(np.float32)
    i = np.arange(width // 2)[None, :].astype(np.float32)
    freq = np.exp(-(2.0 * i / width) * np.log(10000.0))
    ang = pos * freq
    return np.concatenate([np.sin(ang), np.cos(ang)], axis=-1)


def _rprev_kernel(pe_ref, wr_ref, rp_ref):
    pe = pe_ref[...]
    for h in range(H):
        wrh = wr_ref[:, h, :]
        rp_ref[h] = jax.lax.dot_general(
            wrh, pe, (((0,), (1,)), ((), ())),
            preferred_element_type=_F32)


def _proj_kernel(x_ref, s_ref, wq_ref, wk_ref, wv_ref, wg_ref,
                 rp_ref, xlv_ref,
                 q_ref, k_ref, v_ref, g_ref, qr_ref):
    x = x_ref[...]
    ms = jnp.mean(jnp.square(x), axis=-1, keepdims=True)
    xt = x * jax.lax.rsqrt(ms + 1e-6) * s_ref[...]
    q = jnp.dot(xt, wq_ref[...], preferred_element_type=_F32)
    q_ref[...] = q
    k_ref[...] = jnp.dot(xt, wk_ref[...], preferred_element_type=_F32)
    v_ref[...] = jnp.dot(xt, wv_ref[...], preferred_element_type=_F32)
    g_ref[...] = jnp.dot(xt, wg_ref[...], preferred_element_type=_F32)
    for h in range(H):
        qp = q[:, h * DK:(h + 1) * DK] + xlv_ref[h]
        qr_ref[h] = jnp.dot(qp, rp_ref[h],
                            preferred_element_type=_F32)


def _head_kernel(q_ref, k_ref, v_ref, cb_ref, srel_ref, xlu_ref, o_ref):
    for j in range(2):
        lo = j * DK
        k = k_ref[:, lo:lo + DK]
        v = v_ref[:, lo:lo + DK]
        cb = cb_ref[j]
        dots = jax.lax.dot_general(
            k, cb, (((1,), (1,)), ((), ())),
            preferred_element_type=_F32)
        ksq = jnp.sum(k * k, axis=-1, keepdims=True)
        csq = jnp.sum(cb * cb, axis=-1)
        dist = ksq - 2.0 * dots + csq[None, :]
        dmin = jnp.min(dist, axis=-1, keepdims=True)
        sidx = jax.lax.broadcasted_iota(jnp.int32, (T, S), 1)
        z = jnp.min(jnp.where(dist == dmin, sidx, S), axis=-1, keepdims=True)
        delta = (sidx == z).astype(_F32)
        khat = jnp.dot(delta, cb, preferred_element_type=_F32)

        nlist = []
        vlist = []
        for r in range(R):
            dr = delta[r * C:(r + 1) * C]
            vr = v[r * C:(r + 1) * C]
            nlist.append(jnp.sum(dr, axis=0, keepdims=True))
            vlist.append(jax.lax.dot_general(
                dr, vr, (((0,), (0,)), ((), ())),
                preferred_element_type=_F32))

        qc = q_ref[:, lo:lo + DK] + xlu_ref[j]
        scache = jax.lax.dot_general(
            qc, cb, (((1,), (1,)), ((), ())),
            preferred_element_type=_F32) / TAU

        ci = jax.lax.broadcasted_iota(jnp.int32, (C, C), 0)
        cj = jax.lax.broadcasted_iota(jnp.int32, (C, C), 1)
        causal = ci >= cj

        pn = jnp.zeros((1, S), _F32)
        pv = jnp.zeros((S, DV), _F32)
        for r in range(R):
            qcr = qc[r * C:(r + 1) * C]
            kcur = khat[r * C:(r + 1) * C]
            s_cur = (jax.lax.dot_general(
                qcr, kcur, (((1,), (1,)), ((), ())),
                preferred_element_type=_F32)
                + srel_ref[j, r][:, C:]) / TAU
            s_cur = jnp.where(causal, s_cur, -INFTY)
            s_cache = jnp.where(pn > 0.0,
                                scache[r * C:(r + 1) * C], -INFTY)
            mx = jnp.maximum(jnp.max(s_cur, axis=-1),
                             jnp.max(s_cache, axis=-1))
            if r > 0:
                kprev = khat[(r - 1) * C:r * C]
                s_prev = (jax.lax.dot_general(
                    qcr, kprev, (((1,), (1,)), ((), ())),
                    preferred_element_type=_F32)
                    + srel_ref[j, r][:, :C]) / TAU
                mx = jnp.maximum(mx, jnp.max(s_prev, axis=-1))
            m = mx[:, None]
            ac = jnp.exp(s_cur - m)
            ah = jnp.exp(s_cache - m)
            denom = (jnp.sum(ac, axis=-1) + jnp.sum(ah * pn, axis=-1))
            o = (jnp.dot(ac, v[r * C:(r + 1) * C],
                         preferred_element_type=_F32)
                 + jnp.dot(ah, pv, preferred_element_type=_F32))
            if r > 0:
                ap = jnp.exp(s_prev - m)
                denom = denom + jnp.sum(ap, axis=-1)
                o = o + jnp.dot(ap, v[(r - 1) * C:r * C],
                                preferred_element_type=_F32)
            o_ref[r * C:(r + 1) * C, lo:lo + DK] = o / denom[:, None]
            if 1 <= r < R - 1:
                pn = pn + nlist[r - 1]
                pv = pv + vlist[r - 1]


def _out_kernel(o_ref, g_ref, x_ref, wo_ref, y_ref):
    g = g_ref[...]
    og = o_ref[...] * (g * jax.nn.sigmoid(g))
    y_ref[...] = x_ref[...] + jnp.dot(og, wo_ref[...],
                                      preferred_element_type=_F32)


@jax.jit
def kernel(x, ln_scale, w_q, w_k, w_v, w_g, w_r, xl_u, xl_v, codebook, w_o):
    x2 = x.reshape(T, D)
    lns = ln_scale.reshape(1, D)
    pe_rev = jnp.asarray(_sinusoid_np(W, D)[::-1].copy())
    xlv3 = xl_v.reshape(H, 1, DK)
    xlu3 = xl_u.reshape(H, 1, DK)

    rprevT = pl.pallas_call(
        _rprev_kernel,
        grid=(1,),
        in_specs=[
            pl.BlockSpec((W, D), lambda i: (0, 0)),
            pl.BlockSpec((D, H, DK), lambda i: (0, 0, 0)),
        ],
        out_specs=pl.BlockSpec((H, DK, W), lambda i: (0, 0, 0)),
        out_shape=jax.ShapeDtypeStruct((H, DK, W), _F32),
    )(pe_rev, w_r)

    TB = 256
    q, k, v, g, qr = pl.pallas_call(
        _proj_kernel,
        grid=(T // TB,),
        in_specs=[
            pl.BlockSpec((TB, D), lambda i: (i, 0)),
            pl.BlockSpec((1, D), lambda i: (0, 0)),
            pl.BlockSpec((D, H * DK), lambda i: (0, 0)),
            pl.BlockSpec((D, H * DK), lambda i: (0, 0)),
            pl.BlockSpec((D, H * DV), lambda i: (0, 0)),
            pl.BlockSpec((D, H * DV), lambda i: (0, 0)),
            pl.BlockSpec((H, DK, W), lambda i: (0, 0, 0)),
            pl.BlockSpec((H, 1, DK), lambda i: (0, 0, 0)),
        ],
        out_specs=[pl.BlockSpec((TB, H * DK), lambda i: (i, 0))] * 4
        + [pl.BlockSpec((H, TB, W), lambda i: (0, i, 0))],
        out_shape=[jax.ShapeDtypeStruct((T, H * DK), _F32)] * 4
        + [jax.ShapeDtypeStruct((H, T, W), _F32)],
    )(x2, lns, w_q.reshape(D, H * DK), w_k.reshape(D, H * DK),
      w_v.reshape(D, H * DV), w_g.reshape(D, H * DV), rprevT, xlv3)

    arev = qr.reshape(H, R, C, W)
    arev = jnp.pad(arev, ((0, 0), (0, 0), (0, 0), (0, C - 1)))
    flat = arev.reshape(H, R, C * (3 * C - 1))
    flat = flat[:, :, C - 1:C - 1 + C * (3 * C - 2)]
    srel = flat.reshape(H, R, C, 3 * C - 2)[:, :, :, :W]

    o = pl.pallas_call(
        _head_kernel,
        grid=(H // 2,),
        in_specs=[
            pl.BlockSpec((T, 2 * DK), lambda i: (0, i)),
            pl.BlockSpec((T, 2 * DK), lambda i: (0, i)),
            pl.BlockSpec((T, 2 * DV), lambda i: (0, i)),
            pl.BlockSpec((2, S, DK), lambda i: (i, 0, 0)),
            pl.BlockSpec((2, R, C, W), lambda i: (i, 0, 0, 0)),
            pl.BlockSpec((2, 1, DK), lambda i: (i, 0, 0)),
        ],
        out_specs=pl.BlockSpec((T, 2 * DV), lambda i: (0, i)),
        out_shape=jax.ShapeDtypeStruct((T, H * DV), _F32),
    )(q, k, v, codebook, srel, xlu3)

    y = pl.pallas_call(
        _out_kernel,
        grid=(T // TB,),
        in_specs=[
            pl.BlockSpec((TB, H * DV), lambda i: (i, 0)),
            pl.BlockSpec((TB, H * DV), lambda i: (i, 0)),
            pl.BlockSpec((TB, D), lambda i: (i, 0)),
            pl.BlockSpec((H * DV, D), lambda i: (0, 0)),
        ],
        out_specs=pl.BlockSpec((TB, D), lambda i: (i, 0)),
        out_shape=jax.ShapeDtypeStruct((T, D), _F32),
    )(o, g, x2, w_o)

    return y.reshape(B, T, D)


# speedup vs baseline: 3.6667x; 1.1318x over previous
import jax
import jax.numpy as jnp
import numpy as np
from jax.experimental import pallas as pl

B = 1
T = 2048
C = 128
R = T // C
H = 16
DK = 64
DV = 64
D = 1024
S = 512
TAU = float(DK) ** 0.5
INFTY = 1e30
W = 2 * C
WP = 3 * C

_F32 = jnp.float32


def _sinusoid_np(length, width):
    pos = np.arange(length)[:, None].astype(np.float32)
    i = np.arange(width // 2)[None, :].astype(np.float32)
    freq = np.exp(-(2.0 * i / width) * np.log(10000.0))
    ang = pos * freq
    return np.concatenate([np.sin(ang), np.cos(ang)], axis=-1)


def _rprev_kernel(pe_ref, wr_ref, rp_ref):
    pe = pe_ref[...]
    for h in range(H):
        wrh = wr_ref[:, h, :]
        rp_ref[h, :, :W] = jax.lax.dot_general(
            wrh, pe, (((0,), (1,)), ((), ())),
            preferred_element_type=_F32)
        rp_ref[h, :, W:] = jnp.zeros((DK, WP - W), _F32)


def _proj_kernel(x_ref, s_ref, wq_ref, wk_ref, wv_ref, wg_ref,
                 rp_ref, xlv_ref,
                 q_ref, k_ref, v_ref, g_ref, qr_ref):
    x = x_ref[...]
    ms = jnp.mean(jnp.square(x), axis=-1, keepdims=True)
    xt = x * jax.lax.rsqrt(ms + 1e-6) * s_ref[...]
    q = jnp.dot(xt, wq_ref[...], preferred_element_type=_F32)
    q_ref[...] = q
    k_ref[...] = jnp.dot(xt, wk_ref[...], preferred_element_type=_F32)
    v_ref[...] = jnp.dot(xt, wv_ref[...], preferred_element_type=_F32)
    g_ref[...] = jnp.dot(xt, wg_ref[...], preferred_element_type=_F32)
    for h in range(H):
        qp = q[:, h * DK:(h + 1) * DK] + xlv_ref[h]
        qr_ref[h] = jnp.dot(qp, rp_ref[h],
                            preferred_element_type=_F32)


def _head_kernel(q_ref, k_ref, v_ref, cb_ref, srel_ref, xlu_ref, o_ref):
    for j in range(2):
        lo = j * DK
        k = k_ref[:, lo:lo + DK]
        v = v_ref[:, lo:lo + DK]
        cb = cb_ref[j]
        dots = jax.lax.dot_general(
            k, cb, (((1,), (1,)), ((), ())),
            preferred_element_type=_F32)
        ksq = jnp.sum(k * k, axis=-1, keepdims=True)
        csq = jnp.sum(cb * cb, axis=-1)
        dist = ksq - 2.0 * dots + csq[None, :]
        dmin = jnp.min(dist, axis=-1, keepdims=True)
        sidx = jax.lax.broadcasted_iota(jnp.int32, (T, S), 1)
        z = jnp.min(jnp.where(dist == dmin, sidx, S), axis=-1, keepdims=True)
        delta = (sidx == z).astype(_F32)
        khat = jnp.dot(delta, cb, preferred_element_type=_F32)

        nlist = []
        vlist = []
        for r in range(R):
            dr = delta[r * C:(r + 1) * C]
            vr = v[r * C:(r + 1) * C]
            nlist.append(jnp.sum(dr, axis=0, keepdims=True))
            vlist.append(jax.lax.dot_general(
                dr, vr, (((0,), (0,)), ((), ())),
                preferred_element_type=_F32))

        qc = q_ref[:, lo:lo + DK] + xlu_ref[j]
        scache = jax.lax.dot_general(
            qc, cb, (((1,), (1,)), ((), ())),
            preferred_element_type=_F32) / TAU

        ci = jax.lax.broadcasted_iota(jnp.int32, (C, C), 0)
        cj = jax.lax.broadcasted_iota(jnp.int32, (C, C), 1)
        causal = ci >= cj

        pn = jnp.zeros((1, S), _F32)
        pv = jnp.zeros((S, DV), _F32)
        for r in range(R):
            qcr = qc[r * C:(r + 1) * C]
            kcur = khat[r * C:(r + 1) * C]
            s_cur = (jax.lax.dot_general(
                qcr, kcur, (((1,), (1,)), ((), ())),
                preferred_element_type=_F32)
                + srel_ref[j, r][:, C:W]) / TAU
            s_cur = jnp.where(causal, s_cur, -INFTY)
            s_cache = jnp.where(pn > 0.0,
                                scache[r * C:(r + 1) * C], -INFTY)
            mx = jnp.maximum(jnp.max(s_cur, axis=-1),
                             jnp.max(s_cache, axis=-1))
            if r > 0:
                kprev = khat[(r - 1) * C:r * C]
                s_prev = (jax.lax.dot_general(
                    qcr, kprev, (((1,), (1,)), ((), ())),
                    preferred_element_type=_F32)
                    + srel_ref[j, r][:, :C]) / TAU
                mx = jnp.maximum(mx, jnp.max(s_prev, axis=-1))
            m = mx[:, None]
            ac = jnp.exp(s_cur - m)
            ah = jnp.exp(s_cache - m)
            denom = (jnp.sum(ac, axis=-1) + jnp.sum(ah * pn, axis=-1))
            o = (jnp.dot(ac, v[r * C:(r + 1) * C],
                         preferred_element_type=_F32)
                 + jnp.dot(ah, pv, preferred_element_type=_F32))
            if r > 0:
                ap = jnp.exp(s_prev - m)
                denom = denom + jnp.sum(ap, axis=-1)
                o = o + jnp.dot(ap, v[(r - 1) * C:r * C],
                                preferred_element_type=_F32)
            o_ref[r * C:(r + 1) * C, lo:lo + DK] = o / denom[:, None]
            if 1 <= r < R - 1:
                pn = pn + nlist[r - 1]
                pv = pv + vlist[r - 1]


def _out_kernel(o_ref, g_ref, x_ref, wo_ref, y_ref):
    g = g_ref[...]
    og = o_ref[...] * (g * jax.nn.sigmoid(g))
    y_ref[...] = x_ref[...] + jnp.dot(og, wo_ref[...],
                                      preferred_element_type=_F32)


@jax.jit
def kernel(x, ln_scale, w_q, w_k, w_v, w_g, w_r, xl_u, xl_v, codebook, w_o):
    x2 = x.reshape(T, D)
    lns = ln_scale.reshape(1, D)
    pe_rev = jnp.asarray(_sinusoid_np(W, D)[::-1].copy())
    xlv3 = xl_v.reshape(H, 1, DK)
    xlu3 = xl_u.reshape(H, 1, DK)

    rprevT = pl.pallas_call(
        _rprev_kernel,
        grid=(1,),
        in_specs=[
            pl.BlockSpec((W, D), lambda i: (0, 0)),
            pl.BlockSpec((D, H, DK), lambda i: (0, 0, 0)),
        ],
        out_specs=pl.BlockSpec((H, DK, WP), lambda i: (0, 0, 0)),
        out_shape=jax.ShapeDtypeStruct((H, DK, WP), _F32),
    )(pe_rev, w_r)

    TB = 256
    q, k, v, g, qr = pl.pallas_call(
        _proj_kernel,
        grid=(T // TB,),
        in_specs=[
            pl.BlockSpec((TB, D), lambda i: (i, 0)),
            pl.BlockSpec((1, D), lambda i: (0, 0)),
            pl.BlockSpec((D, H * DK), lambda i: (0, 0)),
            pl.BlockSpec((D, H * DK), lambda i: (0, 0)),
            pl.BlockSpec((D, H * DV), lambda i: (0, 0)),
            pl.BlockSpec((D, H * DV), lambda i: (0, 0)),
            pl.BlockSpec((H, DK, WP), lambda i: (0, 0, 0)),
            pl.BlockSpec((H, 1, DK), lambda i: (0, 0, 0)),
        ],
        out_specs=[pl.BlockSpec((TB, H * DK), lambda i: (i, 0))] * 4
        + [pl.BlockSpec((H, TB, WP), lambda i: (0, i, 0))],
        out_shape=[jax.ShapeDtypeStruct((T, H * DK), _F32)] * 4
        + [jax.ShapeDtypeStruct((H, T, WP), _F32)],
    )(x2, lns, w_q.reshape(D, H * DK), w_k.reshape(D, H * DK),
      w_v.reshape(D, H * DV), w_g.reshape(D, H * DV), rprevT, xlv3)

    flat = qr.reshape(H, R, C * WP)
    srel = flat[:, :, C - 1:C - 1 + C * (WP - 1)].reshape(H, R, C, WP - 1)

    o = pl.pallas_call(
        _head_kernel,
        grid=(H // 2,),
        in_specs=[
            pl.BlockSpec((T, 2 * DK), lambda i: (0, i)),
            pl.BlockSpec((T, 2 * DK), lambda i: (0, i)),
            pl.BlockSpec((T, 2 * DV), lambda i: (0, i)),
            pl.BlockSpec((2, S, DK), lambda i: (i, 0, 0)),
            pl.BlockSpec((2, R, C, WP - 1), lambda i: (i, 0, 0, 0)),
            pl.BlockSpec((2, 1, DK), lambda i: (i, 0, 0)),
        ],
        out_specs=pl.BlockSpec((T, 2 * DV), lambda i: (0, i)),
        out_shape=jax.ShapeDtypeStruct((T, H * DV), _F32),
    )(q, k, v, codebook, srel, xlu3)

    y = pl.pallas_call(
        _out_kernel,
        grid=(T // TB,),
        in_specs=[
            pl.BlockSpec((TB, H * DV), lambda i: (i, 0)),
            pl.BlockSpec((TB, H * DV), lambda i: (i, 0)),
            pl.BlockSpec((TB, D), lambda i: (i, 0)),
            pl.BlockSpec((H * DV, D), lambda i: (0, 0)),
        ],
        out_specs=pl.BlockSpec((TB, D), lambda i: (i, 0)),
        out_shape=jax.ShapeDtypeStruct((T, D), _F32),
    )(o, g, x2, w_o)

    return y.reshape(B, T, D)


# speedup vs baseline: 3.8118x; 1.0396x over previous
import jax
import jax.numpy as jnp
import numpy as np
from jax.experimental import pallas as pl

B = 1
T = 2048
C = 128
R = T // C
H = 16
DK = 64
DV = 64
D = 1024
S = 512
TAU = float(DK) ** 0.5
INFTY = 1e30
W = 2 * C
WP = 3 * C

_F32 = jnp.float32


def _sinusoid_np(length, width):
    pos = np.arange(length)[:, None].astype(np.float32)
    i = np.arange(width // 2)[None, :].astype(np.float32)
    freq = np.exp(-(2.0 * i / width) * np.log(10000.0))
    ang = pos * freq
    return np.concatenate([np.sin(ang), np.cos(ang)], axis=-1)


def _rprev_kernel(pe_ref, wr_ref, rp_ref):
    pe = pe_ref[...]
    for h in range(H):
        wrh = wr_ref[:, h, :]
        rp_ref[h, :, :W] = jax.lax.dot_general(
            wrh, pe, (((0,), (1,)), ((), ())),
            preferred_element_type=_F32)
        rp_ref[h, :, W:] = jnp.zeros((DK, WP - W), _F32)


def _proj_kernel(x_ref, s_ref, wq_ref, wk_ref, wv_ref, wg_ref,
                 rp_ref, xlv_ref,
                 q_ref, k_ref, v_ref, g_ref, qr_ref):
    x = x_ref[...]
    ms = jnp.mean(jnp.square(x), axis=-1, keepdims=True)
    xt = x * jax.lax.rsqrt(ms + 1e-6) * s_ref[...]
    q = jnp.dot(xt, wq_ref[...], preferred_element_type=_F32)
    q_ref[...] = q
    k_ref[...] = jnp.dot(xt, wk_ref[...], preferred_element_type=_F32)
    v_ref[...] = jnp.dot(xt, wv_ref[...], preferred_element_type=_F32)
    g_ref[...] = jnp.dot(xt, wg_ref[...], preferred_element_type=_F32)
    for h in range(H):
        qp = q[:, h * DK:(h + 1) * DK] + xlv_ref[h]
        qr_ref[h] = jnp.dot(qp, rp_ref[h],
                            preferred_element_type=_F32)


def _head_kernel(q_ref, k_ref, v_ref, cb_ref, srel_ref, xlu_ref, o_ref):
    for j in range(2):
        lo = j * DK
        k = k_ref[:, lo:lo + DK]
        v = v_ref[:, lo:lo + DK]
        cb = cb_ref[j]
        dots = jax.lax.dot_general(
            k, cb, (((1,), (1,)), ((), ())),
            preferred_element_type=_F32)
        ksq = jnp.sum(k * k, axis=-1, keepdims=True)
        csq = jnp.sum(cb * cb, axis=-1)
        dist = ksq - 2.0 * dots + csq[None, :]
        dmin = jnp.min(dist, axis=-1, keepdims=True)
        sidx = jax.lax.broadcasted_iota(jnp.int32, (T, S), 1)
        z = jnp.min(jnp.where(dist == dmin, sidx, S), axis=-1, keepdims=True)
        delta = (sidx == z).astype(_F32)
        khat = jnp.dot(delta, cb, preferred_element_type=_F32)

        nlist = []
        vlist = []
        for r in range(R):
            dr = delta[r * C:(r + 1) * C]
            vr = v[r * C:(r + 1) * C]
            nlist.append(jnp.sum(dr, axis=0, keepdims=True))
            vlist.append(jax.lax.dot_general(
                dr, vr, (((0,), (0,)), ((), ())),
                preferred_element_type=_F32))

        qc = q_ref[:, lo:lo + DK] + xlu_ref[j]
        scache = jax.lax.dot_general(
            qc, cb, (((1,), (1,)), ((), ())),
            preferred_element_type=_F32) / TAU

        ci = jax.lax.broadcasted_iota(jnp.int32, (C, C), 0)
        cj = jax.lax.broadcasted_iota(jnp.int32, (C, C), 1)
        causal = ci >= cj
        wi = jax.lax.broadcasted_iota(jnp.int32, (C, W), 0)
        wj = jax.lax.broadcasted_iota(jnp.int32, (C, W), 1)
        wmask = wj <= wi + C

        pn = jnp.zeros((1, S), _F32)
        pv = jnp.zeros((S, DV), _F32)
        for r in range(R):
            qcr = qc[r * C:(r + 1) * C]
            s_cache = jnp.where(pn > 0.0,
                                scache[r * C:(r + 1) * C], -INFTY)
            if r > 0:
                kw = khat[(r - 1) * C:(r + 1) * C]
                vw = v[(r - 1) * C:(r + 1) * C]
                s_win = (jax.lax.dot_general(
                    qcr, kw, (((1,), (1,)), ((), ())),
                    preferred_element_type=_F32)
                    + srel_ref[j, r][:, :W]) / TAU
                s_win = jnp.where(wmask, s_win, -INFTY)
            else:
                kw = khat[0:C]
                vw = v[0:C]
                s_win = (jax.lax.dot_general(
                    qcr, kw, (((1,), (1,)), ((), ())),
                    preferred_element_type=_F32)
                    + srel_ref[j, 0][:, C:W]) / TAU
                s_win = jnp.where(causal, s_win, -INFTY)
            m = jnp.maximum(jnp.max(s_win, axis=-1),
                            jnp.max(s_cache, axis=-1))[:, None]
            aw = jnp.exp(s_win - m)
            ah = jnp.exp(s_cache - m)
            denom = jnp.sum(aw, axis=-1) + jnp.sum(ah * pn, axis=-1)
            o = (jnp.dot(aw, vw, preferred_element_type=_F32)
                 + jnp.dot(ah, pv, preferred_element_type=_F32))
            o_ref[r * C:(r + 1) * C, lo:lo + DK] = o / denom[:, None]
            if 1 <= r < R - 1:
                pn = pn + nlist[r - 1]
                pv = pv + vlist[r - 1]


def _out_kernel(o_ref, g_ref, x_ref, wo_ref, y_ref):
    g = g_ref[...]
    og = o_ref[...] * (g * jax.nn.sigmoid(g))
    y_ref[...] = x_ref[...] + jnp.dot(og, wo_ref[...],
                                      preferred_element_type=_F32)


@jax.jit
def kernel(x, ln_scale, w_q, w_k, w_v, w_g, w_r, xl_u, xl_v, codebook, w_o):
    x2 = x.reshape(T, D)
    lns = ln_scale.reshape(1, D)
    pe_rev = jnp.asarray(_sinusoid_np(W, D)[::-1].copy())
    xlv3 = xl_v.reshape(H, 1, DK)
    xlu3 = xl_u.reshape(H, 1, DK)

    rprevT = pl.pallas_call(
        _rprev_kernel,
        grid=(1,),
        in_specs=[
            pl.BlockSpec((W, D), lambda i: (0, 0)),
            pl.BlockSpec((D, H, DK), lambda i: (0, 0, 0)),
        ],
        out_specs=pl.BlockSpec((H, DK, WP), lambda i: (0, 0, 0)),
        out_shape=jax.ShapeDtypeStruct((H, DK, WP), _F32),
    )(pe_rev, w_r)

    TB = 256
    q, k, v, g, qr = pl.pallas_call(
        _proj_kernel,
        grid=(T // TB,),
        in_specs=[
            pl.BlockSpec((TB, D), lambda i: (i, 0)),
            pl.BlockSpec((1, D), lambda i: (0, 0)),
            pl.BlockSpec((D, H * DK), lambda i: (0, 0)),
            pl.BlockSpec((D, H * DK), lambda i: (0, 0)),
            pl.BlockSpec((D, H * DV), lambda i: (0, 0)),
            pl.BlockSpec((D, H * DV), lambda i: (0, 0)),
            pl.BlockSpec((H, DK, WP), lambda i: (0, 0, 0)),
            pl.BlockSpec((H, 1, DK), lambda i: (0, 0, 0)),
        ],
        out_specs=[pl.BlockSpec((TB, H * DK), lambda i: (i, 0))] * 4
        + [pl.BlockSpec((H, TB, WP), lambda i: (0, i, 0))],
        out_shape=[jax.ShapeDtypeStruct((T, H * DK), _F32)] * 4
        + [jax.ShapeDtypeStruct((H, T, WP), _F32)],
    )(x2, lns, w_q.reshape(D, H * DK), w_k.reshape(D, H * DK),
      w_v.reshape(D, H * DV), w_g.reshape(D, H * DV), rprevT, xlv3)

    flat = qr.reshape(H, R, C * WP)
    srel = flat[:, :, C - 1:C - 1 + C * (WP - 1)].reshape(H, R, C, WP - 1)

    o = pl.pallas_call(
        _head_kernel,
        grid=(H // 2,),
        in_specs=[
            pl.BlockSpec((T, 2 * DK), lambda i: (0, i)),
            pl.BlockSpec((T, 2 * DK), lambda i: (0, i)),
            pl.BlockSpec((T, 2 * DV), lambda i: (0, i)),
            pl.BlockSpec((2, S, DK), lambda i: (i, 0, 0)),
            pl.BlockSpec((2, R, C, WP - 1), lambda i: (i, 0, 0, 0)),
            pl.BlockSpec((2, 1, DK), lambda i: (i, 0, 0)),
        ],
        out_specs=pl.BlockSpec((T, 2 * DV), lambda i: (0, i)),
        out_shape=jax.ShapeDtypeStruct((T, H * DV), _F32),
    )(q, k, v, codebook, srel, xlu3)

    y = pl.pallas_call(
        _out_kernel,
        grid=(T // TB,),
        in_specs=[
            pl.BlockSpec((TB, H * DV), lambda i: (i, 0)),
            pl.BlockSpec((TB, H * DV), lambda i: (i, 0)),
            pl.BlockSpec((TB, D), lambda i: (i, 0)),
            pl.BlockSpec((H * DV, D), lambda i: (0, 0)),
        ],
        out_specs=pl.BlockSpec((TB, D), lambda i: (i, 0)),
        out_shape=jax.ShapeDtypeStruct((T, D), _F32),
    )(o, g, x2, w_o)

    return y.reshape(B, T, D)


# speedup vs baseline: 4.1009x; 1.0758x over previous
import jax
import jax.numpy as jnp
import numpy as np
from jax.experimental import pallas as pl

B = 1
T = 2048
C = 128
R = T // C
H = 16
DK = 64
DV = 64
D = 1024
S = 512
TAU = float(DK) ** 0.5
INFTY = 1e30
W = 2 * C
WP = 3 * C

_F32 = jnp.float32


def _sinusoid_np(length, width):
    pos = np.arange(length)[:, None].astype(np.float32)
    i = np.arange(width // 2)[None, :].astype(np.float32)
    freq = np.exp(-(2.0 * i / width) * np.log(10000.0))
    ang = pos * freq
    return np.concatenate([np.sin(ang), np.cos(ang)], axis=-1)


def _rprev_kernel(pe_ref, wr_ref, rp_ref):
    pe = pe_ref[...]
    for h in range(H):
        wrh = wr_ref[:, h, :]
        rp_ref[h, :, :W] = jax.lax.dot_general(
            wrh, pe, (((0,), (1,)), ((), ())),
            preferred_element_type=_F32)
        rp_ref[h, :, W:] = jnp.zeros((DK, WP - W), _F32)


def _proj_kernel(x_ref, s_ref, wq_ref, wk_ref, wv_ref, wg_ref,
                 rp_ref, xlv_ref,
                 q_ref, k_ref, v_ref, g_ref, qr_ref):
    x = x_ref[...]
    ms = jnp.mean(jnp.square(x), axis=-1, keepdims=True)
    xt = x * jax.lax.rsqrt(ms + 1e-6) * s_ref[...]
    q = jnp.dot(xt, wq_ref[...], preferred_element_type=_F32)
    q_ref[...] = q
    k_ref[...] = jnp.dot(xt, wk_ref[...], preferred_element_type=_F32)
    v_ref[...] = jnp.dot(xt, wv_ref[...], preferred_element_type=_F32)
    g_ref[...] = jnp.dot(xt, wg_ref[...], preferred_element_type=_F32)
    for h in range(H):
        qp = q[:, h * DK:(h + 1) * DK] + xlv_ref[h]
        qr_ref[h] = jnp.dot(qp, rp_ref[h],
                            preferred_element_type=_F32
                            ).astype(jnp.bfloat16)


def _head_kernel(q_ref, k_ref, v_ref, cb_ref, srel_ref, xlu_ref, o_ref):
    for j in range(2):
        lo = j * DK
        k = k_ref[:, lo:lo + DK]
        v = v_ref[:, lo:lo + DK]
        cb = cb_ref[j]
        dots = jax.lax.dot_general(
            k, cb, (((1,), (1,)), ((), ())),
            preferred_element_type=_F32)
        ksq = jnp.sum(k * k, axis=-1, keepdims=True)
        csq = jnp.sum(cb * cb, axis=-1)
        dist = ksq - 2.0 * dots + csq[None, :]
        dmin = jnp.min(dist, axis=-1, keepdims=True)
        sidx = jax.lax.broadcasted_iota(jnp.int32, (T, S), 1)
        z = jnp.min(jnp.where(dist == dmin, sidx, S), axis=-1, keepdims=True)
        delta = (sidx == z).astype(_F32)
        khat = jnp.dot(delta, cb, preferred_element_type=_F32)

        nlist = []
        vlist = []
        for r in range(R):
            dr = delta[r * C:(r + 1) * C]
            vr = v[r * C:(r + 1) * C]
            nlist.append(jnp.sum(dr, axis=0, keepdims=True))
            vlist.append(jax.lax.dot_general(
                dr, vr, (((0,), (0,)), ((), ())),
                preferred_element_type=_F32))

        qc = q_ref[:, lo:lo + DK] + xlu_ref[j]
        scache = jax.lax.dot_general(
            qc, cb, (((1,), (1,)), ((), ())),
            preferred_element_type=_F32) / TAU

        ci = jax.lax.broadcasted_iota(jnp.int32, (C, C), 0)
        cj = jax.lax.broadcasted_iota(jnp.int32, (C, C), 1)
        causal = ci >= cj
        wi = jax.lax.broadcasted_iota(jnp.int32, (C, W), 0)
        wj = jax.lax.broadcasted_iota(jnp.int32, (C, W), 1)
        wmask = wj <= wi + C

        pn = jnp.zeros((1, S), _F32)
        pv = jnp.zeros((S, DV), _F32)
        for r in range(R):
            qcr = qc[r * C:(r + 1) * C]
            s_cache = jnp.where(pn > 0.0,
                                scache[r * C:(r + 1) * C], -INFTY)
            if r > 0:
                kw = khat[(r - 1) * C:(r + 1) * C]
                vw = v[(r - 1) * C:(r + 1) * C]
                s_win = (jax.lax.dot_general(
                    qcr, kw, (((1,), (1,)), ((), ())),
                    preferred_element_type=_F32)
                    + srel_ref[j, r][:, :W]) / TAU
                s_win = jnp.where(wmask, s_win, -INFTY)
            else:
                kw = khat[0:C]
                vw = v[0:C]
                s_win = (jax.lax.dot_general(
                    qcr, kw, (((1,), (1,)), ((), ())),
                    preferred_element_type=_F32)
                    + srel_ref[j, 0][:, C:W]) / TAU
                s_win = jnp.where(causal, s_win, -INFTY)
            m = jnp.maximum(jnp.max(s_win, axis=-1),
                            jnp.max(s_cache, axis=-1))[:, None]
            aw = jnp.exp(s_win - m)
            ah = jnp.exp(s_cache - m)
            denom = jnp.sum(aw, axis=-1) + jnp.sum(ah * pn, axis=-1)
            o = (jnp.dot(aw, vw, preferred_element_type=_F32)
                 + jnp.dot(ah, pv, preferred_element_type=_F32))
            o_ref[r * C:(r + 1) * C, lo:lo + DK] = o / denom[:, None]
            if 1 <= r < R - 1:
                pn = pn + nlist[r - 1]
                pv = pv + vlist[r - 1]


def _out_kernel(o_ref, g_ref, x_ref, wo_ref, y_ref):
    g = g_ref[...]
    og = o_ref[...] * (g * jax.nn.sigmoid(g))
    y_ref[...] = x_ref[...] + jnp.dot(og, wo_ref[...],
                                      preferred_element_type=_F32)


@jax.jit
def kernel(x, ln_scale, w_q, w_k, w_v, w_g, w_r, xl_u, xl_v, codebook, w_o):
    x2 = x.reshape(T, D)
    lns = ln_scale.reshape(1, D)
    pe_rev = jnp.asarray(_sinusoid_np(W, D)[::-1].copy())
    xlv3 = xl_v.reshape(H, 1, DK)
    xlu3 = xl_u.reshape(H, 1, DK)

    rprevT = pl.pallas_call(
        _rprev_kernel,
        grid=(1,),
        in_specs=[
            pl.BlockSpec((W, D), lambda i: (0, 0)),
            pl.BlockSpec((D, H, DK), lambda i: (0, 0, 0)),
        ],
        out_specs=pl.BlockSpec((H, DK, WP), lambda i: (0, 0, 0)),
        out_shape=jax.ShapeDtypeStruct((H, DK, WP), _F32),
    )(pe_rev, w_r)

    TB = 256
    q, k, v, g, qr = pl.pallas_call(
        _proj_kernel,
        grid=(T // TB,),
        in_specs=[
            pl.BlockSpec((TB, D), lambda i: (i, 0)),
            pl.BlockSpec((1, D), lambda i: (0, 0)),
            pl.BlockSpec((D, H * DK), lambda i: (0, 0)),
            pl.BlockSpec((D, H * DK), lambda i: (0, 0)),
            pl.BlockSpec((D, H * DV), lambda i: (0, 0)),
            pl.BlockSpec((D, H * DV), lambda i: (0, 0)),
            pl.BlockSpec((H, DK, WP), lambda i: (0, 0, 0)),
            pl.BlockSpec((H, 1, DK), lambda i: (0, 0, 0)),
        ],
        out_specs=[pl.BlockSpec((TB, H * DK), lambda i: (i, 0))] * 4
        + [pl.BlockSpec((H, TB, WP), lambda i: (0, i, 0))],
        out_shape=[jax.ShapeDtypeStruct((T, H * DK), _F32)] * 4
        + [jax.ShapeDtypeStruct((H, T, WP), jnp.bfloat16)],
    )(x2, lns, w_q.reshape(D, H * DK), w_k.reshape(D, H * DK),
      w_v.reshape(D, H * DV), w_g.reshape(D, H * DV), rprevT, xlv3)

    flat = qr.reshape(H, R, C * WP)
    srel = flat[:, :, C - 1:C - 1 + C * (WP - 1)].reshape(H, R, C, WP - 1)

    o = pl.pallas_call(
        _head_kernel,
        grid=(H // 2,),
        in_specs=[
            pl.BlockSpec((T, 2 * DK), lambda i: (0, i)),
            pl.BlockSpec((T, 2 * DK), lambda i: (0, i)),
            pl.BlockSpec((T, 2 * DV), lambda i: (0, i)),
            pl.BlockSpec((2, S, DK), lambda i: (i, 0, 0)),
            pl.BlockSpec((2, R, C, WP - 1), lambda i: (i, 0, 0, 0)),
            pl.BlockSpec((2, 1, DK), lambda i: (i, 0, 0)),
        ],
        out_specs=pl.BlockSpec((T, 2 * DV), lambda i: (0, i)),
        out_shape=jax.ShapeDtypeStruct((T, H * DV), _F32),
    )(q, k, v, codebook, srel, xlu3)

    y = pl.pallas_call(
        _out_kernel,
        grid=(T // TB,),
        in_specs=[
            pl.BlockSpec((TB, H * DV), lambda i: (i, 0)),
            pl.BlockSpec((TB, H * DV), lambda i: (i, 0)),
            pl.BlockSpec((TB, D), lambda i: (i, 0)),
            pl.BlockSpec((H * DV, D), lambda i: (0, 0)),
        ],
        out_specs=pl.BlockSpec((TB, D), lambda i: (i, 0)),
        out_shape=jax.ShapeDtypeStruct((T, D), _F32),
    )(o, g, x2, w_o)

    return y.reshape(B, T, D)


# speedup vs baseline: 4.1298x; 1.0070x over previous
import jax
import jax.numpy as jnp
import numpy as np
from jax.experimental import pallas as pl
from jax.experimental.pallas import tpu as pltpu

B = 1
T = 2048
C = 128
R = T // C
H = 16
DK = 64
DV = 64
D = 1024
S = 512
TAU = float(DK) ** 0.5
INFTY = 1e30
W = 2 * C
WP = 3 * C

_F32 = jnp.float32


def _sinusoid_np(length, width):
    pos = np.arange(length)[:, None].astype(np.float32)
    i = np.arange(width // 2)[None, :].astype(np.float32)
    freq = np.exp(-(2.0 * i / width) * np.log(10000.0))
    ang = pos * freq
    return np.concatenate([np.sin(ang), np.cos(ang)], axis=-1)


def _proj_kernel(x_ref, s_ref, wq_ref, wk_ref, wv_ref, wg_ref,
                 pe_ref, wr_ref, xlv_ref,
                 q_ref, k_ref, v_ref, g_ref, qr_ref, rp_ref):
    @pl.when(pl.program_id(0) == 0)
    def _init_rel_table():
        pe = pe_ref[...]
        for h in range(H):
            wrh = wr_ref[:, h, :]
            rp_ref[h, :, :W] = jax.lax.dot_general(
                wrh, pe, (((0,), (1,)), ((), ())),
                preferred_element_type=_F32)
            rp_ref[h, :, W:] = jnp.zeros((DK, WP - W), _F32)

    x = x_ref[...]
    ms = jnp.mean(jnp.square(x), axis=-1, keepdims=True)
    xt = x * jax.lax.rsqrt(ms + 1e-6) * s_ref[...]
    q = jnp.dot(xt, wq_ref[...], preferred_element_type=_F32)
    q_ref[...] = q
    k_ref[...] = jnp.dot(xt, wk_ref[...], preferred_element_type=_F32)
    v_ref[...] = jnp.dot(xt, wv_ref[...], preferred_element_type=_F32)
    g_ref[...] = jnp.dot(xt, wg_ref[...], preferred_element_type=_F32)
    for h in range(H):
        qp = q[:, h * DK:(h + 1) * DK] + xlv_ref[h]
        qr_ref[h] = jnp.dot(qp, rp_ref[h],
                            preferred_element_type=_F32
                            ).astype(jnp.bfloat16)


def _head_kernel(q_ref, k_ref, v_ref, cb_ref, srel_ref, xlu_ref, o_ref):
    for j in range(2):
        lo = j * DK
        k = k_ref[:, lo:lo + DK]
        v = v_ref[:, lo:lo + DK]
        cb = cb_ref[j]
        dots = jax.lax.dot_general(
            k, cb, (((1,), (1,)), ((), ())),
            preferred_element_type=_F32)
        ksq = jnp.sum(k * k, axis=-1, keepdims=True)
        csq = jnp.sum(cb * cb, axis=-1)
        dist = ksq - 2.0 * dots + csq[None, :]
        dmin = jnp.min(dist, axis=-1, keepdims=True)
        sidx = jax.lax.broadcasted_iota(jnp.int32, (T, S), 1)
        z = jnp.min(jnp.where(dist == dmin, sidx, S), axis=-1, keepdims=True)
        delta = (sidx == z).astype(_F32)
        khat = jnp.dot(delta, cb, preferred_element_type=_F32)

        nlist = []
        vlist = []
        for r in range(R - 2):
            dr = delta[r * C:(r + 1) * C]
            vr = v[r * C:(r + 1) * C]
            nlist.append(jnp.sum(dr, axis=0, keepdims=True))
            vlist.append(jax.lax.dot_general(
                dr, vr, (((0,), (0,)), ((), ())),
                preferred_element_type=_F32))

        qc = q_ref[:, lo:lo + DK] + xlu_ref[j]
        scache = jax.lax.dot_general(
            qc, cb, (((1,), (1,)), ((), ())),
            preferred_element_type=_F32) / TAU

        ci = jax.lax.broadcasted_iota(jnp.int32, (C, C), 0)
        cj = jax.lax.broadcasted_iota(jnp.int32, (C, C), 1)
        causal = ci >= cj
        wi = jax.lax.broadcasted_iota(jnp.int32, (C, W), 0)
        wj = jax.lax.broadcasted_iota(jnp.int32, (C, W), 1)
        wmask = wj <= wi + C

        pn = jnp.zeros((1, S), _F32)
        pv = jnp.zeros((S, DV), _F32)
        for r in range(R):
            qcr = qc[r * C:(r + 1) * C]
            s_cache = jnp.where(pn > 0.0,
                                scache[r * C:(r + 1) * C], -INFTY)
            if r > 0:
                kw = khat[(r - 1) * C:(r + 1) * C]
                vw = v[(r - 1) * C:(r + 1) * C]
                s_win = (jax.lax.dot_general(
                    qcr, kw, (((1,), (1,)), ((), ())),
                    preferred_element_type=_F32)
                    + srel_ref[j, r][:, :W]) / TAU
                s_win = jnp.where(wmask, s_win, -INFTY)
            else:
                kw = khat[0:C]
                vw = v[0:C]
                s_win = (jax.lax.dot_general(
                    qcr, kw, (((1,), (1,)), ((), ())),
                    preferred_element_type=_F32)
                    + srel_ref[j, 0][:, C:W]) / TAU
                s_win = jnp.where(causal, s_win, -INFTY)
            m = jnp.maximum(jnp.max(s_win, axis=-1),
                            jnp.max(s_cache, axis=-1))[:, None]
            aw = jnp.exp(s_win - m)
            ah = jnp.exp(s_cache - m)
            denom = jnp.sum(aw, axis=-1) + jnp.sum(ah * pn, axis=-1)
            o = (jnp.dot(aw, vw, preferred_element_type=_F32)
                 + jnp.dot(ah, pv, preferred_element_type=_F32))
            o_ref[r * C:(r + 1) * C, lo:lo + DK] = o / denom[:, None]
            if 1 <= r < R - 1:
                pn = pn + nlist[r - 1]
                pv = pv + vlist[r - 1]


def _out_kernel(o_ref, g_ref, x_ref, wo_ref, y_ref):
    g = g_ref[...]
    og = o_ref[...] * (g * jax.nn.sigmoid(g))
    y_ref[...] = x_ref[...] + jnp.dot(og, wo_ref[...],
                                      preferred_element_type=_F32)


@jax.jit
def kernel(x, ln_scale, w_q, w_k, w_v, w_g, w_r, xl_u, xl_v, codebook, w_o):
    x2 = x.reshape(T, D)
    lns = ln_scale.reshape(1, D)
    pe_rev = jnp.asarray(_sinusoid_np(W, D)[::-1].copy())
    xlv3 = xl_v.reshape(H, 1, DK)
    xlu3 = xl_u.reshape(H, 1, DK)

    TB = 256
    q, k, v, g, qr = pl.pallas_call(
        _proj_kernel,
        grid=(T // TB,),
        in_specs=[
            pl.BlockSpec((TB, D), lambda i: (i, 0)),
            pl.BlockSpec((1, D), lambda i: (0, 0)),
            pl.BlockSpec((D, H * DK), lambda i: (0, 0)),
            pl.BlockSpec((D, H * DK), lambda i: (0, 0)),
            pl.BlockSpec((D, H * DV), lambda i: (0, 0)),
            pl.BlockSpec((D, H * DV), lambda i: (0, 0)),
            pl.BlockSpec((W, D), lambda i: (0, 0)),
            pl.BlockSpec((D, H, DK), lambda i: (0, 0, 0)),
            pl.BlockSpec((H, 1, DK), lambda i: (0, 0, 0)),
        ],
        out_specs=[pl.BlockSpec((TB, H * DK), lambda i: (i, 0))] * 4
        + [pl.BlockSpec((H, TB, WP), lambda i: (0, i, 0))],
        out_shape=[jax.ShapeDtypeStruct((T, H * DK), _F32)] * 4
        + [jax.ShapeDtypeStruct((H, T, WP), jnp.bfloat16)],
        scratch_shapes=[pltpu.VMEM((H, DK, WP), _F32)],
    )(x2, lns, w_q.reshape(D, H * DK), w_k.reshape(D, H * DK),
      w_v.reshape(D, H * DV), w_g.reshape(D, H * DV), pe_rev, w_r, xlv3)

    flat = qr.reshape(H, R, C * WP)
    srel = flat[:, :, C - 1:C - 1 + C * (WP - 1)].reshape(H, R, C, WP - 1)

    o = pl.pallas_call(
        _head_kernel,
        grid=(H // 2,),
        in_specs=[
            pl.BlockSpec((T, 2 * DK), lambda i: (0, i)),
            pl.BlockSpec((T, 2 * DK), lambda i: (0, i)),
            pl.BlockSpec((T, 2 * DV), lambda i: (0, i)),
            pl.BlockSpec((2, S, DK), lambda i: (i, 0, 0)),
            pl.BlockSpec((2, R, C, WP - 1), lambda i: (i, 0, 0, 0)),
            pl.BlockSpec((2, 1, DK), lambda i: (i, 0, 0)),
        ],
        out_specs=pl.BlockSpec((T, 2 * DV), lambda i: (0, i)),
        out_shape=jax.ShapeDtypeStruct((T, H * DV), _F32),
    )(q, k, v, codebook, srel, xlu3)

    y = pl.pallas_call(
        _out_kernel,
        grid=(T // TB,),
        in_specs=[
            pl.BlockSpec((TB, H * DV), lambda i: (i, 0)),
            pl.BlockSpec((TB, H * DV), lambda i: (i, 0)),
            pl.BlockSpec((TB, D), lambda i: (i, 0)),
            pl.BlockSpec((H * DV, D), lambda i: (0, 0)),
        ],
        out_specs=pl.BlockSpec((TB, D), lambda i: (i, 0)),
        out_shape=jax.ShapeDtypeStruct((T, D), _F32),
    )(o, g, x2, w_o)

    return y.reshape(B, T, D)
